# baseline (device time: 40935 ns/iter reference)
import jax
import jax.numpy as jnp
from jax import lax
from jax.experimental import pallas as pl
from jax.experimental.pallas import tpu as pltpu

N_CHUNKS = 16


def kernel(partial, resid, gamma):
    _, m, d = partial.shape
    half = m // 2
    rc = half // N_CHUNKS

    def body(p_ref, r_ref, g_ref, o_ref, pbuf, rbuf, xcomm, obuf,
             p_sems, r_sem, store_sems, x_send_sems, x_recv_sems,
             y_send_sems, y_recv_sems):
        my_x = lax.axis_index("x")
        my_y = lax.axis_index("y")
        x_peer = (1 - my_x, my_y)
        y_peer = (my_x, 1 - my_y)
        half_off = my_y * half

        my_rows = pl.ds(half_off, half)
        pcopies = []
        for c in range(N_CHUNKS):
            lrows = pl.ds(c * rc, rc)
            rows = pl.ds(half_off + c * rc, rc)
            cp = pltpu.make_async_copy(
                p_ref.at[0, rows, :], pbuf.at[lrows, :], p_sems.at[c]
            )
            cp.start()
            pcopies.append(cp)
        rcopy = pltpu.make_async_copy(r_ref.at[my_rows, :], rbuf, r_sem)
        rcopy.start()

        barrier_sem = pltpu.get_barrier_semaphore()
        for peer in (x_peer, y_peer):
            pl.semaphore_signal(
                barrier_sem, inc=1, device_id=peer,
                device_id_type=pl.DeviceIdType.MESH,
            )
        pl.semaphore_wait(barrier_sem, 2)

        x_rdmas = []
        for c in range(N_CHUNKS):
            lrows = pl.ds(c * rc, rc)
            pcopies[c].wait()
            rdma = pltpu.make_async_remote_copy(
                src_ref=pbuf.at[lrows, :],
                dst_ref=xcomm.at[lrows, :],
                send_sem=x_send_sems.at[c],
                recv_sem=x_recv_sems.at[c],
                device_id=x_peer,
                device_id_type=pl.DeviceIdType.MESH,
            )
            rdma.start()
            x_rdmas.append(rdma)

        rcopy.wait()

        y_rdmas = []
        stores = []
        for c in range(N_CHUNKS):
            lrows = pl.ds(c * rc, rc)
            rows = pl.ds(half_off + c * rc, rc)
            x_rdmas[c].wait_recv()
            y = pbuf[lrows, :] + xcomm[lrows, :] + rbuf[lrows, :]
            rms = jnp.sqrt(jnp.mean(y * y, axis=-1, keepdims=True) + 1e-6)
            obuf[lrows, :] = y / rms * g_ref[0][None, :]
            st = pltpu.make_async_copy(
                obuf.at[lrows, :], o_ref.at[rows, :], store_sems.at[c]
            )
            st.start()
            stores.append(st)
            rdma = pltpu.make_async_remote_copy(
                src_ref=obuf.at[lrows, :],
                dst_ref=o_ref.at[rows, :],
                send_sem=y_send_sems.at[c],
                recv_sem=y_recv_sems.at[c],
                device_id=y_peer,
                device_id_type=pl.DeviceIdType.MESH,
            )
            rdma.start()
            y_rdmas.append(rdma)

        for c in range(N_CHUNKS):
            stores[c].wait()
            x_rdmas[c].wait_send()
            y_rdmas[c].wait_send()
            y_rdmas[c].wait_recv()

    return pl.pallas_call(
        body,
        out_shape=jax.ShapeDtypeStruct((m, d), jnp.float32),
        in_specs=[
            pl.BlockSpec(memory_space=pl.ANY),
            pl.BlockSpec(memory_space=pl.ANY),
            pl.BlockSpec(memory_space=pltpu.VMEM),
        ],
        out_specs=pl.BlockSpec(memory_space=pl.ANY),
        scratch_shapes=[
            pltpu.VMEM((half, d), jnp.float32),
            pltpu.VMEM((half, d), jnp.float32),
            pltpu.VMEM((half, d), jnp.float32),
            pltpu.VMEM((half, d), jnp.float32),
            pltpu.SemaphoreType.DMA((N_CHUNKS,)),
            pltpu.SemaphoreType.DMA,
            pltpu.SemaphoreType.DMA((N_CHUNKS,)),
            pltpu.SemaphoreType.DMA((N_CHUNKS,)),
            pltpu.SemaphoreType.DMA((N_CHUNKS,)),
            pltpu.SemaphoreType.DMA((N_CHUNKS,)),
            pltpu.SemaphoreType.DMA((N_CHUNKS,)),
        ],
        compiler_params=pltpu.CompilerParams(collective_id=0),
    )(partial, resid, gamma.reshape(1, d))


# device time: 39175 ns/iter; 1.0449x vs baseline; 1.0449x over previous
import jax
import jax.numpy as jnp
from jax import lax
from jax.experimental import pallas as pl
from jax.experimental.pallas import tpu as pltpu

N_CHUNKS = 8


def kernel(partial, resid, gamma):
    _, m, d = partial.shape
    half = m // 2
    rc = half // N_CHUNKS

    def body(p_ref, r_ref, g_ref, o_ref, pbuf, rbuf, xcomm, obuf,
             p_sem, r_sem, store_sems, x_send_sems, x_recv_sems,
             y_send_sems, y_recv_sems):
        my_x = lax.axis_index("x")
        my_y = lax.axis_index("y")
        x_peer = (1 - my_x, my_y)
        y_peer = (my_x, 1 - my_y)
        half_off = my_y * half

        my_rows = pl.ds(half_off, half)
        pcopy = pltpu.make_async_copy(p_ref.at[0, my_rows, :], pbuf, p_sem)
        pcopy.start()
        rcopy = pltpu.make_async_copy(r_ref.at[my_rows, :], rbuf, r_sem)
        rcopy.start()

        barrier_sem = pltpu.get_barrier_semaphore()
        for peer in (x_peer, y_peer):
            pl.semaphore_signal(
                barrier_sem, inc=1, device_id=peer,
                device_id_type=pl.DeviceIdType.MESH,
            )
        pl.semaphore_wait(barrier_sem, 2)
        pcopy.wait()

        x_rdmas = []
        for c in range(N_CHUNKS):
            lrows = pl.ds(c * rc, rc)
            rdma = pltpu.make_async_remote_copy(
                src_ref=pbuf.at[lrows, :],
                dst_ref=xcomm.at[lrows, :],
                send_sem=x_send_sems.at[c],
                recv_sem=x_recv_sems.at[c],
                device_id=x_peer,
                device_id_type=pl.DeviceIdType.MESH,
            )
            rdma.start()
            x_rdmas.append(rdma)

        rcopy.wait()

        y_rdmas = []
        stores = []
        for c in range(N_CHUNKS):
            lrows = pl.ds(c * rc, rc)
            rows = pl.ds(half_off + c * rc, rc)
            x_rdmas[c].wait_recv()
            y = pbuf[lrows, :] + xcomm[lrows, :] + rbuf[lrows, :]
            rms = jnp.sqrt(jnp.mean(y * y, axis=-1, keepdims=True) + 1e-6)
            obuf[lrows, :] = y / rms * g_ref[0][None, :]
            st = pltpu.make_async_copy(
                obuf.at[lrows, :], o_ref.at[rows, :], store_sems.at[c]
            )
            st.start()
            stores.append(st)
            rdma = pltpu.make_async_remote_copy(
                src_ref=obuf.at[lrows, :],
                dst_ref=o_ref.at[rows, :],
                send_sem=y_send_sems.at[c],
                recv_sem=y_recv_sems.at[c],
                device_id=y_peer,
                device_id_type=pl.DeviceIdType.MESH,
            )
            rdma.start()
            y_rdmas.append(rdma)

        for c in range(N_CHUNKS):
            stores[c].wait()
            x_rdmas[c].wait_send()
            y_rdmas[c].wait_send()
            y_rdmas[c].wait_recv()

    return pl.pallas_call(
        body,
        out_shape=jax.ShapeDtypeStruct((m, d), jnp.float32),
        in_specs=[
            pl.BlockSpec(memory_space=pl.ANY),
            pl.BlockSpec(memory_space=pl.ANY),
            pl.BlockSpec(memory_space=pltpu.VMEM),
        ],
        out_specs=pl.BlockSpec(memory_space=pl.ANY),
        scratch_shapes=[
            pltpu.VMEM((half, d), jnp.float32),
            pltpu.VMEM((half, d), jnp.float32),
            pltpu.VMEM((half, d), jnp.float32),
            pltpu.VMEM((half, d), jnp.float32),
            pltpu.SemaphoreType.DMA,
            pltpu.SemaphoreType.DMA,
            pltpu.SemaphoreType.DMA((N_CHUNKS,)),
            pltpu.SemaphoreType.DMA((N_CHUNKS,)),
            pltpu.SemaphoreType.DMA((N_CHUNKS,)),
            pltpu.SemaphoreType.DMA((N_CHUNKS,)),
            pltpu.SemaphoreType.DMA((N_CHUNKS,)),
        ],
        compiler_params=pltpu.CompilerParams(collective_id=0),
    )(partial, resid, gamma.reshape(1, d))
